# baseline (speedup 1.0000x reference)
import functools

import jax
import jax.numpy as jnp
from jax import lax
from jax.experimental import pallas as pl
from jax.experimental.pallas import tpu as pltpu
from jax.experimental.pallas import tpu_sc as plsc

B = 4096
F = 26
VOCAB = 1000000
WIDE_DIM = 10
DEEP_DIM = 16

NC = 2
NS = 16
NW = NC * NS

NI = B * F
CHUNK = 128
ROWS_PER_TILE = NI // NW
CHUNKS_PER_TILE = ROWS_PER_TILE // CHUNK
LANES = 16

W16_ROWS = VOCAB * WIDE_DIM // DEEP_DIM
D_IN = F * DEEP_DIM


def _gather_kernel(idx_hbm, deep_hbm, wide16_hbm, deep_out, wa_out, wb_out,
                   idx_v, r0_v, r1_v, buf0, buf1, sem_i, sem_d, sem_w):
    wid = lax.axis_index("s") * NC + lax.axis_index("c")
    pltpu.async_copy(idx_hbm.at[wid], idx_v, sem_i).wait()

    hs = []
    for j in range(CHUNKS_PER_TILE):
        hs.append(pltpu.async_copy(
            deep_hbm.at[idx_v.at[j]], buf0.at[pl.ds(j * CHUNK, CHUNK)],
            sem_d))

    @pl.loop(0, CHUNKS_PER_TILE)
    def _(j):
        @pl.loop(0, CHUNK // LANES)
        def _(k):
            sl = pl.ds(k * LANES, LANES)
            t = idx_v[j, sl] * WIDE_DIM
            r0 = lax.shift_right_logical(t, 4)
            r0_v[j, sl] = r0
            r1_v[j, sl] = jnp.minimum(r0 + 1, W16_ROWS - 1)

    for h in hs:
        h.wait()
    base = wid * ROWS_PER_TILE
    pltpu.async_copy(buf0, deep_out.at[pl.ds(base, ROWS_PER_TILE)],
                     sem_d).wait()

    hs = []
    for j in range(CHUNKS_PER_TILE):
        hs.append(pltpu.async_copy(
            wide16_hbm.at[r0_v.at[j]], buf0.at[pl.ds(j * CHUNK, CHUNK)],
            sem_w))
        hs.append(pltpu.async_copy(
            wide16_hbm.at[r1_v.at[j]], buf1.at[pl.ds(j * CHUNK, CHUNK)],
            sem_w))
    for h in hs:
        h.wait()
    pltpu.async_copy(buf0, wa_out.at[pl.ds(base, ROWS_PER_TILE)],
                     sem_d).wait()
    pltpu.async_copy(buf1, wb_out.at[pl.ds(base, ROWS_PER_TILE)],
                     sem_w).wait()


def _sc_gather(x_flat3d, deep_table, wide16):
    mesh = plsc.VectorSubcoreMesh(core_axis_name="c", subcore_axis_name="s",
                                  num_cores=NC, num_subcores=NS)
    kern = pl.kernel(
        _gather_kernel,
        out_type=[
            jax.ShapeDtypeStruct((NI, DEEP_DIM), jnp.float32),
            jax.ShapeDtypeStruct((NI, DEEP_DIM), jnp.float32),
            jax.ShapeDtypeStruct((NI, DEEP_DIM), jnp.float32),
        ],
        mesh=mesh,
        scratch_types=[
            pltpu.VMEM((CHUNKS_PER_TILE, CHUNK), jnp.int32),
            pltpu.VMEM((CHUNKS_PER_TILE, CHUNK), jnp.int32),
            pltpu.VMEM((CHUNKS_PER_TILE, CHUNK), jnp.int32),
            pltpu.VMEM((ROWS_PER_TILE, DEEP_DIM), jnp.float32),
            pltpu.VMEM((ROWS_PER_TILE, DEEP_DIM), jnp.float32),
            pltpu.SemaphoreType.DMA,
            pltpu.SemaphoreType.DMA,
            pltpu.SemaphoreType.DMA,
        ],
        compiler_params=pltpu.CompilerParams(use_tc_tiling_on_sc=False),
    )
    return kern(x_flat3d, deep_table, wide16)


def _mlp_kernel(x_ref, deep_ref, wa_ref, wb_ref, g_ref, b_ref, w1_ref, b1_ref,
                w2_ref, b2_ref, w3_ref, b3_ref, w4_ref, b4_ref, out_ref):
    dot = functools.partial(lax.dot_general,
                            dimension_numbers=(((1,), (0,)), ((), ())),
                            preferred_element_type=jnp.float32,
                            precision=lax.Precision.HIGHEST)
    d = deep_ref[...]
    mu = jnp.mean(d, axis=-1, keepdims=True)
    c = d - mu
    var = jnp.mean(c * c, axis=-1, keepdims=True)
    h = c * lax.rsqrt(var + 1e-5) * g_ref[...] + b_ref[...]
    h = jnp.maximum(dot(h, w1_ref[...]) + b1_ref[...], 0.0)
    h = jnp.maximum(dot(h, w2_ref[...]) + b2_ref[...], 0.0)
    h = jnp.maximum(dot(h, w3_ref[...]) + b3_ref[...], 0.0)
    dnn = dot(h, w4_ref[...]) + b4_ref[...]

    xb = x_ref[...]
    s26 = ((xb * WIDE_DIM) & (LANES - 1)).astype(jnp.float32)
    fc = lax.broadcasted_iota(jnp.int32, (F, D_IN), 1) // DEEP_DIM
    fr = lax.broadcasted_iota(jnp.int32, (F, D_IN), 0)
    expand = (fc == fr).astype(jnp.float32)
    s416 = dot(s26, expand)
    lane16 = (lax.broadcasted_iota(jnp.int32, (1, D_IN), 1)
              % DEEP_DIM).astype(jnp.float32)
    a = wa_ref[...]
    bwin = wb_ref[...]
    mask_a = (lane16 >= s416) & (lane16 < s416 + WIDE_DIM)
    mask_b = lane16 < s416 - (DEEP_DIM - WIDE_DIM)
    wide = jnp.sum(jnp.where(mask_a, a, 0.0) + jnp.where(mask_b, bwin, 0.0),
                   axis=-1, keepdims=True)
    out_ref[...] = jax.nn.sigmoid(dnn + wide)


def _tc_mlp(x, deep_g, wa, wb, ln_g, ln_b, W1, b1, W2, b2, W3, b3, W4, b4):
    bb = 512
    grid = (B // bb,)

    def full(shape):
        return pl.BlockSpec(shape, lambda i: (0,) * len(shape))

    return pl.pallas_call(
        _mlp_kernel,
        grid=grid,
        in_specs=[
            pl.BlockSpec((bb, F), lambda i: (i, 0)),
            pl.BlockSpec((bb, D_IN), lambda i: (i, 0)),
            pl.BlockSpec((bb, D_IN), lambda i: (i, 0)),
            pl.BlockSpec((bb, D_IN), lambda i: (i, 0)),
            full((1, D_IN)),
            full((1, D_IN)),
            full(W1.shape),
            full((1, W1.shape[1])),
            full(W2.shape),
            full((1, W2.shape[1])),
            full(W3.shape),
            full((1, W3.shape[1])),
            full(W4.shape),
            full((1, W4.shape[1])),
        ],
        out_specs=pl.BlockSpec((bb, 1), lambda i: (i, 0)),
        out_shape=jax.ShapeDtypeStruct((B, 1), jnp.float32),
    )(x, deep_g, wa, wb, ln_g.reshape(1, -1), ln_b.reshape(1, -1),
      W1, b1.reshape(1, -1), W2, b2.reshape(1, -1),
      W3, b3.reshape(1, -1), W4, b4.reshape(1, -1))


@jax.jit
def kernel(x, wide_table, deep_table, ln_g, ln_b, W1, b1, W2, b2, W3, b3,
           W4, b4):
    x_flat3d = x.reshape(NW, CHUNKS_PER_TILE, CHUNK)
    wide16 = wide_table.reshape(W16_ROWS, DEEP_DIM)
    deep_g, wa, wb = _sc_gather(x_flat3d, deep_table, wide16)
    deep_g = deep_g.reshape(B, D_IN)
    wa = wa.reshape(B, D_IN)
    wb = wb.reshape(B, D_IN)
    return _tc_mlp(x, deep_g, wa, wb, ln_g, ln_b, W1, b1, W2, b2, W3, b3,
                   W4, b4)
